# baseline (speedup 1.0000x reference)
import dataclasses
import functools

import jax
import jax.numpy as jnp
from jax.experimental import pallas as pl
from jax.experimental.pallas import tpu as pltpu
from jax.experimental.pallas import tpu_sc as plsc

_NC = 2
_NS = 16
_NW = _NC * _NS
_LANES = 16

_CHUNK = 8
_NBUF = 3
_FAST_CHUNK = 64


def _gather_rows(x, valid_idx, n_rows, d):
    per_w = n_rows // _NW
    n_chunks = per_w // _CHUNK
    mesh = plsc.VectorSubcoreMesh(core_axis_name="core",
                                  subcore_axis_name="subcore")
    cp = pltpu.CompilerParams()
    if "needs_layout_passes" in pltpu.CompilerParams.__dataclass_fields__:
        cp = dataclasses.replace(cp, needs_layout_passes=False)

    @functools.partial(
        pl.kernel,
        out_type=jax.ShapeDtypeStruct((n_rows, d), x.dtype),
        mesh=mesh,
        compiler_params=cp,
        scratch_types=[
            pltpu.VMEM((per_w,), jnp.int32),
            pltpu.VMEM((_NBUF, _CHUNK, d), x.dtype),
            pltpu.SemaphoreType.DMA,
            pltpu.SemaphoreType.DMA,
        ],
    )
    def gather_kernel(x_hbm, i_hbm, o_hbm, idx_v, buf, sem_in, sem_out):
        wid = jax.lax.axis_index("subcore") * _NC + jax.lax.axis_index("core")
        base = wid * per_w
        pltpu.sync_copy(i_hbm.at[pl.ds(base, per_w)], idx_v)

        lanes = jax.lax.iota(jnp.int32, _LANES)
        contig = None
        for k in range(per_w // _LANES):
            v = idx_v[pl.ds(k * _LANES, _LANES)]
            ok = jnp.all(v == base + k * _LANES + lanes)
            contig = ok if contig is None else jnp.logical_and(contig, ok)

        @pl.when(contig)
        def _fast():
            def blockcopy(c):
                return pltpu.make_async_copy(
                    x_hbm.at[pl.ds(base + c * _FAST_CHUNK, _FAST_CHUNK)],
                    o_hbm.at[pl.ds(base + c * _FAST_CHUNK, _FAST_CHUNK)],
                    sem_in)

            for c in range(per_w // _FAST_CHUNK):
                blockcopy(c).start()
            for c in range(per_w // _FAST_CHUNK):
                blockcopy(c).wait()

        @pl.when(jnp.logical_not(contig))
        def _slow():
            def gather(c):
                return pltpu.make_async_copy(
                    x_hbm.at[idx_v.at[pl.ds(c * _CHUNK, _CHUNK)]],
                    buf.at[c % _NBUF], sem_in)

            def writeback(c):
                return pltpu.make_async_copy(
                    buf.at[c % _NBUF],
                    o_hbm.at[pl.ds(base + c * _CHUNK, _CHUNK)], sem_out)

            for c in range(min(_NBUF - 1, n_chunks)):
                gather(c).start()
            pending_wb = 0
            for c in range(n_chunks):
                gather(c).wait()
                writeback(c).start()
                pending_wb += 1
                nxt = c + _NBUF - 1
                if nxt < n_chunks:
                    if pending_wb > _NBUF - 2:
                        writeback(c - 1).wait()
                        pending_wb -= 1
                    gather(nxt).start()
            for _ in range(pending_wb):
                writeback(n_chunks - 1).wait()

    return gather_kernel(x, valid_idx)


def kernel(x, valid_idx):
    n_rows = valid_idx.shape[0]
    d = x.shape[1]
    return _gather_rows(x, valid_idx, n_rows, d)


# speedup vs baseline: 31.1408x; 31.1408x over previous
import dataclasses
import functools

import jax
import jax.numpy as jnp
from jax.experimental import pallas as pl
from jax.experimental.pallas import tpu as pltpu
from jax.experimental.pallas import tpu_sc as plsc

_NC = 2
_NS = 16
_NW = _NC * _NS
_LANES = 16

_CHUNK = 8
_NBUF = 3
_FAST_CHUNK = 64


def _gather_rows(x, valid_idx, n_rows, d):
    per_w = n_rows // _NW
    n_chunks = per_w // _CHUNK
    mesh = plsc.VectorSubcoreMesh(core_axis_name="core",
                                  subcore_axis_name="subcore")
    cp = pltpu.CompilerParams()
    if "needs_layout_passes" in pltpu.CompilerParams.__dataclass_fields__:
        cp = dataclasses.replace(cp, needs_layout_passes=False)

    @functools.partial(
        pl.kernel,
        out_type=jax.ShapeDtypeStruct((n_rows, d), x.dtype),
        mesh=mesh,
        compiler_params=cp,
        scratch_types=[
            pltpu.VMEM((per_w,), jnp.int32),
            pltpu.VMEM((_NBUF, _CHUNK, d), x.dtype),
            pltpu.SemaphoreType.DMA,
            pltpu.SemaphoreType.DMA,
        ],
    )
    def gather_kernel(x_hbm, i_hbm, o_hbm, idx_v, buf, sem_in, sem_out):
        wid = jax.lax.axis_index("subcore") * _NC + jax.lax.axis_index("core")
        base = wid * per_w
        pltpu.sync_copy(i_hbm.at[pl.ds(base, per_w)], idx_v)

        lanes = jax.lax.iota(jnp.int32, _LANES)
        contig = None
        for k in range(per_w // _LANES):
            v = idx_v[pl.ds(k * _LANES, _LANES)]
            ok = jnp.all(v == base + k * _LANES + lanes)
            contig = ok if contig is None else jnp.logical_and(contig, ok)

        def ring_pipeline(gather):
            def writeback(c):
                return pltpu.make_async_copy(
                    buf.at[c % _NBUF],
                    o_hbm.at[pl.ds(base + c * _CHUNK, _CHUNK)], sem_out)

            for c in range(min(_NBUF - 1, n_chunks)):
                gather(c).start()
            pending_wb = 0
            for c in range(n_chunks):
                gather(c).wait()
                writeback(c).start()
                pending_wb += 1
                nxt = c + _NBUF - 1
                if nxt < n_chunks:
                    if pending_wb > _NBUF - 2:
                        writeback(c - 1).wait()
                        pending_wb -= 1
                    gather(nxt).start()
            for _ in range(pending_wb):
                writeback(n_chunks - 1).wait()

        @pl.when(contig)
        def _fast():
            ring_pipeline(lambda c: pltpu.make_async_copy(
                x_hbm.at[pl.ds(base + c * _CHUNK, _CHUNK)],
                buf.at[c % _NBUF], sem_in))

        @pl.when(jnp.logical_not(contig))
        def _slow():
            ring_pipeline(lambda c: pltpu.make_async_copy(
                x_hbm.at[idx_v.at[pl.ds(c * _CHUNK, _CHUNK)]],
                buf.at[c % _NBUF], sem_in))

    return gather_kernel(x, valid_idx)


def kernel(x, valid_idx):
    n_rows = valid_idx.shape[0]
    d = x.shape[1]
    return _gather_rows(x, valid_idx, n_rows, d)
